# baseline (speedup 1.0000x reference)
import functools

import jax
import jax.numpy as jnp
import numpy as np
from jax.experimental import pallas as pl
from jax.experimental.pallas import tpu as pltpu
from jax.sharding import Mesh, PartitionSpec as P

_K = 30
_TILE = 512

_HI = jax.lax.Precision.HIGHEST


def _body(xt_ref, xa_ref, o_ref, *, n, k):
    xt = xt_ref[0]
    xa = xa_ref[0]

    xx_r = jnp.sum(xt * xt, axis=1, keepdims=True)
    xx_c = jnp.sum(xa * xa, axis=0, keepdims=True)
    g = jax.lax.dot_general(xt.astype(jnp.bfloat16), xa.astype(jnp.bfloat16),
                            (((1,), (0,)), ((), ())),
                            preferred_element_type=jnp.float32)
    d = xx_r + xx_c - 2.0 * g

    kb = jax.lax.bitcast_convert_type(d, jnp.int32)
    imin = jnp.int32(-2147483648)
    key = jnp.where(kb >= 0, kb, imin - kb)
    ones = jnp.ones((n, 1), jnp.float32)

    def count_f(sel):
        return jax.lax.dot_general(sel, ones, (((1,), (0,)), ((), ())),
                                   preferred_element_type=jnp.float32)

    def count_le(mask):
        return count_f(jnp.where(mask, jnp.float32(1), jnp.float32(0)))

    kf = jnp.float32(k)
    rows = d.shape[0]

    def bstep(_, carry):
        lo, hi = carry
        mid = (lo >> 1) + (hi >> 1) + (lo & hi & jnp.int32(1))
        cnt = count_le(key <= mid)
        ge = cnt >= kf
        return jnp.where(ge, lo, mid), jnp.where(ge, mid, hi)

    lo0 = jnp.full((rows, 1), imin, jnp.int32)
    hi0 = jnp.full((rows, 1), jnp.int32(2147483647), jnp.int32)
    _, t = jax.lax.fori_loop(0, 32, bstep, (lo0, hi0))

    less = key < t
    eq = key == t
    need = kf - count_le(less)
    eqf = jnp.where(eq, jnp.float32(1), jnp.float32(0))
    iota = jax.lax.broadcasted_iota(jnp.int32, d.shape, 1)

    def jstep(_, carry):
        lo, hi = carry
        mid = (lo + hi) >> 1
        cnt = count_f(jnp.where(iota <= mid, eqf, jnp.float32(0)))
        ge = cnt >= need
        return jnp.where(ge, lo, mid), jnp.where(ge, mid, hi)

    jlo0 = jnp.full((rows, 1), jnp.int32(-1), jnp.int32)
    jhi0 = jnp.full((rows, 1), jnp.int32(n - 1), jnp.int32)
    _, jthr = jax.lax.fori_loop(0, 12, jstep, (jlo0, jhi0))
    m = jnp.where(less | (eq & (iota <= jthr)), jnp.float32(1),
                  jnp.float32(0))

    sp = jax.lax.dot_general(m, xa, (((1,), (1,)), ((), ())),
                             preferred_element_type=jnp.float32,
                             precision=_HI)
    kf = jnp.float32(k)
    mu0 = sp[:, 0:1] / kf
    mu1 = sp[:, 1:2] / kf
    mu2 = sp[:, 2:3] / kf

    def cent(row, mu):
        cc = xa[row:row + 1] - mu
        return cc.astype(jnp.bfloat16).astype(jnp.float32)

    c0 = cent(0, mu0)
    c1 = cent(1, mu1)
    c2 = cent(2, mu2)

    cm0 = c0 * m
    cm1 = c1 * m
    cm2 = c2 * m
    onesf = jnp.ones((n, 1), jnp.float32)

    def covsum(u, v):
        return jax.lax.dot_general(u * v, onesf, (((1,), (0,)), ((), ())),
                                   preferred_element_type=jnp.float32,
                                   precision=_HI)

    a = covsum(cm0, c0)
    b = covsum(cm0, c1)
    c = covsum(cm0, c2)
    dN = covsum(cm1, c1)
    e = covsum(cm1, c2)
    f = covsum(cm2, c2)

    q = (a + dN + f) / 3.0
    p1 = b * b + c * c + e * e
    aq = a - q
    dq = dN - q
    fq = f - q
    p2 = aq * aq + dq * dq + fq * fq + 2.0 * p1
    p = jnp.sqrt(p2 / 6.0)
    lam = q - 2.0 * p

    def newton(_, lam):
        al = a - lam
        dl = dN - lam
        fl = f - lam
        m00 = dl * fl - e * e
        m11 = al * fl - c * c
        m22 = al * dl - b * b
        fval = al * m00 - b * (b * fl - e * c) + c * (b * e - dl * c)
        fprime = -(m00 + m11 + m22)
        denom = fprime * fprime + jnp.float32(1e-38)
        return lam - fval * fprime / denom

    lam = jax.lax.fori_loop(0, 15, newton, lam)

    r0a, r0b, r0c = a - lam, b, c
    r1a, r1b, r1c = b, dN - lam, e
    r2a, r2b, r2c = c, e, f - lam

    def cross(ua, ub, uc, va, vb, vc):
        return (ub * vc - uc * vb, uc * va - ua * vc, ua * vb - ub * va)

    w0 = cross(r0a, r0b, r0c, r1a, r1b, r1c)
    w1 = cross(r0a, r0b, r0c, r2a, r2b, r2c)
    w2 = cross(r1a, r1b, r1c, r2a, r2b, r2c)
    n0sq = w0[0] * w0[0] + w0[1] * w0[1] + w0[2] * w0[2]
    n1sq = w1[0] * w1[0] + w1[1] * w1[1] + w1[2] * w1[2]
    n2sq = w2[0] * w2[0] + w2[1] * w2[1] + w2[2] * w2[2]

    use1 = n1sq > n0sq
    bsq = jnp.where(use1, n1sq, n0sq)
    vx = jnp.where(use1, w1[0], w0[0])
    vy = jnp.where(use1, w1[1], w0[1])
    vz = jnp.where(use1, w1[2], w0[2])
    use2 = n2sq > bsq
    bsq = jnp.where(use2, n2sq, bsq)
    vx = jnp.where(use2, w2[0], vx)
    vy = jnp.where(use2, w2[1], vy)
    vz = jnp.where(use2, w2[2], vz)

    inv = jax.lax.rsqrt(jnp.maximum(bsq, jnp.float32(1e-38)))
    vx = vx * inv
    vy = vy * inv
    vz = vz * inv

    dot = -(xt[:, 0:1] * vx + xt[:, 1:2] * vy + xt[:, 2:3] * vz)
    flip = jnp.where(dot < 0.0, jnp.float32(-1.0), jnp.float32(1.0))
    o_ref[0] = jnp.concatenate([vx * flip, vy * flip, vz * flip], axis=1)


def _estimate(x):
    B, C, N = x.shape
    xt = jnp.transpose(x, (0, 2, 1))
    grid = (B, N // _TILE)
    normals = pl.pallas_call(
        functools.partial(_body, n=N, k=_K),
        grid=grid,
        in_specs=[
            pl.BlockSpec((1, _TILE, C), lambda b, r: (b, r, 0)),
            pl.BlockSpec((1, C, N), lambda b, r: (b, 0, 0)),
        ],
        out_specs=pl.BlockSpec((1, _TILE, C), lambda b, r: (b, r, 0)),
        out_shape=jax.ShapeDtypeStruct((B, N, C), jnp.float32),
    )(xt, x)
    return jnp.concatenate([x, jnp.transpose(normals, (0, 2, 1))], axis=1)


@jax.jit
def kernel(x):
    devs = jax.devices()[:2]
    if len(devs) == 2 and x.shape[0] % 2 == 0:
        mesh = Mesh(np.array(devs), ("d",))
        return jax.shard_map(_estimate, mesh=mesh, in_specs=P("d"),
                             out_specs=P("d"), check_vma=False)(x)
    return _estimate(x)


# speedup vs baseline: 1.3778x; 1.3778x over previous
import functools

import jax
import jax.numpy as jnp
import numpy as np
from jax.experimental import pallas as pl
from jax.experimental.pallas import tpu as pltpu
from jax.sharding import Mesh, PartitionSpec as P

_K = 30
_TILE = 512

_HI = jax.lax.Precision.HIGHEST


def _body(xt_ref, xa_ref, o_ref, *, n, k):
    xt = xt_ref[0]
    xa = xa_ref[0]

    xx_r = jnp.sum(xt * xt, axis=1, keepdims=True)
    xx_c = jnp.sum(xa * xa, axis=0, keepdims=True)
    g = jax.lax.dot_general(xt.astype(jnp.bfloat16), xa.astype(jnp.bfloat16),
                            (((1,), (0,)), ((), ())),
                            preferred_element_type=jnp.float32)
    d = xx_r + xx_c - 2.0 * g

    kb = jax.lax.bitcast_convert_type(d, jnp.int32)
    imin = jnp.int32(-2147483648)
    key = jnp.where(kb >= 0, kb, imin - kb)
    ones = jnp.ones((n, 1), jnp.float32)

    def count_f(sel):
        return jax.lax.dot_general(sel, ones, (((1,), (0,)), ((), ())),
                                   preferred_element_type=jnp.float32)

    def count_le(mask):
        return count_f(jnp.where(mask, jnp.float32(1), jnp.float32(0)))

    kf = jnp.float32(k)
    rows = d.shape[0]

    def bstep(_, carry):
        lo, hi = carry
        mid = (lo >> 1) + (hi >> 1) + (lo & hi & jnp.int32(1))
        cnt = count_le(key <= mid)
        ge = cnt >= kf
        return jnp.where(ge, lo, mid), jnp.where(ge, mid, hi)

    lo0 = jnp.full((rows, 1), imin, jnp.int32)
    hi0 = jnp.full((rows, 1), jnp.int32(2147483647), jnp.int32)
    _, t = jax.lax.fori_loop(0, 32, bstep, (lo0, hi0))

    less = key < t
    eq = key == t
    need = kf - count_le(less)
    eqf = jnp.where(eq, jnp.float32(1), jnp.float32(0))
    iota = jax.lax.broadcasted_iota(jnp.int32, d.shape, 1)

    def jstep(_, carry):
        lo, hi = carry
        mid = (lo + hi) >> 1
        cnt = count_f(jnp.where(iota <= mid, eqf, jnp.float32(0)))
        ge = cnt >= need
        return jnp.where(ge, lo, mid), jnp.where(ge, mid, hi)

    jlo0 = jnp.full((rows, 1), jnp.int32(-1), jnp.int32)
    jhi0 = jnp.full((rows, 1), jnp.int32(n - 1), jnp.int32)
    _, jthr = jax.lax.fori_loop(0, 12, jstep, (jlo0, jhi0))
    m = jnp.where(less | (eq & (iota <= jthr)), jnp.float32(1),
                  jnp.float32(0))

    sp = jax.lax.dot_general(m, xa, (((1,), (1,)), ((), ())),
                             preferred_element_type=jnp.float32,
                             precision=_HI)
    kf = jnp.float32(k)
    mu0 = sp[:, 0:1] / kf
    mu1 = sp[:, 1:2] / kf
    mu2 = sp[:, 2:3] / kf

    def cent(row, mu):
        cc = xa[row:row + 1] - mu
        return cc.astype(jnp.bfloat16).astype(jnp.float32)

    c0 = cent(0, mu0)
    c1 = cent(1, mu1)
    c2 = cent(2, mu2)

    def covsum(u, v):
        return jnp.sum(jnp.where(m > 0.0, u * v, 0.0), axis=1,
                       keepdims=True)

    a = covsum(c0, c0)
    b = covsum(c0, c1)
    c = covsum(c0, c2)
    dN = covsum(c1, c1)
    e = covsum(c1, c2)
    f = covsum(c2, c2)

    q = (a + dN + f) / 3.0
    p1 = b * b + c * c + e * e
    aq = a - q
    dq = dN - q
    fq = f - q
    p2 = aq * aq + dq * dq + fq * fq + 2.0 * p1
    p = jnp.sqrt(p2 / 6.0)
    lam = q - 2.0 * p

    def newton(_, lam):
        al = a - lam
        dl = dN - lam
        fl = f - lam
        m00 = dl * fl - e * e
        m11 = al * fl - c * c
        m22 = al * dl - b * b
        fval = al * m00 - b * (b * fl - e * c) + c * (b * e - dl * c)
        fprime = -(m00 + m11 + m22)
        denom = fprime * fprime + jnp.float32(1e-38)
        return lam - fval * fprime / denom

    lam = jax.lax.fori_loop(0, 15, newton, lam)

    r0a, r0b, r0c = a - lam, b, c
    r1a, r1b, r1c = b, dN - lam, e
    r2a, r2b, r2c = c, e, f - lam

    def cross(ua, ub, uc, va, vb, vc):
        return (ub * vc - uc * vb, uc * va - ua * vc, ua * vb - ub * va)

    w0 = cross(r0a, r0b, r0c, r1a, r1b, r1c)
    w1 = cross(r0a, r0b, r0c, r2a, r2b, r2c)
    w2 = cross(r1a, r1b, r1c, r2a, r2b, r2c)
    n0sq = w0[0] * w0[0] + w0[1] * w0[1] + w0[2] * w0[2]
    n1sq = w1[0] * w1[0] + w1[1] * w1[1] + w1[2] * w1[2]
    n2sq = w2[0] * w2[0] + w2[1] * w2[1] + w2[2] * w2[2]

    use1 = n1sq > n0sq
    bsq = jnp.where(use1, n1sq, n0sq)
    vx = jnp.where(use1, w1[0], w0[0])
    vy = jnp.where(use1, w1[1], w0[1])
    vz = jnp.where(use1, w1[2], w0[2])
    use2 = n2sq > bsq
    bsq = jnp.where(use2, n2sq, bsq)
    vx = jnp.where(use2, w2[0], vx)
    vy = jnp.where(use2, w2[1], vy)
    vz = jnp.where(use2, w2[2], vz)

    inv = jax.lax.rsqrt(jnp.maximum(bsq, jnp.float32(1e-38)))
    vx = vx * inv
    vy = vy * inv
    vz = vz * inv

    dot = -(xt[:, 0:1] * vx + xt[:, 1:2] * vy + xt[:, 2:3] * vz)
    flip = jnp.where(dot < 0.0, jnp.float32(-1.0), jnp.float32(1.0))
    o_ref[0] = jnp.concatenate([vx * flip, vy * flip, vz * flip], axis=1)


def _estimate(x):
    B, C, N = x.shape
    xt = jnp.transpose(x, (0, 2, 1))
    grid = (B, N // _TILE)
    normals = pl.pallas_call(
        functools.partial(_body, n=N, k=_K),
        grid=grid,
        in_specs=[
            pl.BlockSpec((1, _TILE, C), lambda b, r: (b, r, 0)),
            pl.BlockSpec((1, C, N), lambda b, r: (b, 0, 0)),
        ],
        out_specs=pl.BlockSpec((1, _TILE, C), lambda b, r: (b, r, 0)),
        out_shape=jax.ShapeDtypeStruct((B, N, C), jnp.float32),
    )(xt, x)
    return jnp.concatenate([x, jnp.transpose(normals, (0, 2, 1))], axis=1)


@jax.jit
def kernel(x):
    devs = jax.devices()[:2]
    if len(devs) == 2 and x.shape[0] % 2 == 0:
        mesh = Mesh(np.array(devs), ("d",))
        return jax.shard_map(_estimate, mesh=mesh, in_specs=P("d"),
                             out_specs=P("d"), check_vma=False)(x)
    return _estimate(x)
